# Initial kernel scaffold; baseline (speedup 1.0000x reference)
import jax
import jax.numpy as jnp
from jax.experimental import pallas as pl


def kernel(token_id, attn_mask, gate_mask, token_weight):
    raise NotImplementedError("write your pallas kernel here")


# speedup vs baseline: 4.6266x; 4.6266x over previous
import jax
import jax.numpy as jnp
from jax.experimental import pallas as pl

_K = 32
_L = 200
_NEG_INF = float("-inf")


def _topk_body(tw_ref, tid_ref, gate_ref, attn_ref, w_ref, tid_out_ref, attn_out_ref):
    tw = tw_ref[...]
    gate = gate_ref[...]
    tid = tid_ref[...]
    attn = attn_ref[...]
    l, c = tw.shape

    pos = jax.lax.broadcasted_iota(jnp.int32, (l, c), 0)
    s = jnp.sum(gate, axis=0, keepdims=True)
    need = s < _K
    keep = (pos >= 1) & (pos <= _K)
    unmask = (gate != 0) | (keep & need)
    twm = jnp.where(unmask, tw, _NEG_INF)

    packed = pos * 65536 + attn * 32768 + tid

    kiota = jax.lax.broadcasted_iota(jnp.int32, (_K, c), 0)
    vals = jnp.zeros((_K, c), jnp.float32)
    keys = jnp.zeros((_K, c), jnp.int32)
    big = jnp.int32(1 << 30)
    for k in range(_K):
        m = jnp.max(twm, axis=0, keepdims=True)
        eq = twm == m
        minp = jnp.min(jnp.where(eq, packed, big), axis=0, keepdims=True)
        sel = packed == minp
        twm = jnp.where(sel, _NEG_INF, twm)
        vals = jnp.where(kiota == k, m, vals)
        keys = jnp.where(kiota == k, minp, keys)

    e = jnp.exp(vals - vals[0:1, :])
    w = e / jnp.sum(e, axis=0, keepdims=True)

    w_ref[...] = w
    tid_out_ref[...] = keys & 32767
    attn_out_ref[...] = (keys >> 15) & 1


def kernel(token_id, attn_mask, gate_mask, token_weight):
    b, l = token_weight.shape
    c = 512
    grid = (b // c,)

    tw_t = token_weight.T
    tid_t = token_id.T
    gate_t = gate_mask.T
    attn_t = attn_mask.T

    in_spec = pl.BlockSpec((l, c), lambda j: (0, j))
    out_spec = pl.BlockSpec((_K, c), lambda j: (0, j))

    w_t, tid_o, attn_o = pl.pallas_call(
        _topk_body,
        grid=grid,
        in_specs=[in_spec, in_spec, in_spec, in_spec],
        out_specs=[out_spec, out_spec, out_spec],
        out_shape=[
            jax.ShapeDtypeStruct((_K, b), jnp.float32),
            jax.ShapeDtypeStruct((_K, b), jnp.int32),
            jax.ShapeDtypeStruct((_K, b), jnp.int32),
        ],
    )(tw_t, tid_t, gate_t, attn_t)

    return (tid_o.T, attn_o.T, w_t.T)
